# Initial kernel scaffold; baseline (speedup 1.0000x reference)
import jax
import jax.numpy as jnp
from jax.experimental import pallas as pl


def kernel(word_table, char_table, W3, b3, W4, b4, W5, b5, q1, q2, q1_len, q2_len, q1_char, q2_char):
    raise NotImplementedError("write your pallas kernel here")


# speedup vs baseline: 3.0225x; 3.0225x over previous
import functools

import numpy as np
import jax
import jax.numpy as jnp
from jax import lax
from jax.experimental import pallas as pl
from jax.experimental.pallas import tpu as pltpu
from jax.experimental.pallas import tpu_sc as plsc

B, T, C = 1024, 50, 16
V, D = 1000000, 64
CV, CD = 128, 32
NF = 50
N = B * T


_NC, _NS = 2, 16
_NW = _NC * _NS
_BPW = N // _NW


def _sc_gather_call(word_table, idx1, idx2):
    mesh = plsc.VectorSubcoreMesh(core_axis_name="c", subcore_axis_name="s")

    @functools.partial(
        pl.kernel,
        mesh=mesh,
        compiler_params=pltpu.CompilerParams(use_tc_tiling_on_sc=False),
        out_type=(
            jax.ShapeDtypeStruct((N, D), jnp.float32),
            jax.ShapeDtypeStruct((N, D), jnp.float32),
        ),
        scratch_types=[
            pltpu.VMEM((_BPW,), jnp.int32),
            pltpu.VMEM((_BPW, D), jnp.float32),
            pltpu.SemaphoreType.DMA,
        ],
    )
    def k(table_hbm, idx1_hbm, idx2_hbm, out1_hbm, out2_hbm, idx_v, rows_v, sem):
        wid = lax.axis_index("s") * _NC + lax.axis_index("c")
        base = wid * _BPW
        for idx_hbm, out_hbm in ((idx1_hbm, out1_hbm), (idx2_hbm, out2_hbm)):
            pltpu.sync_copy(idx_hbm.at[pl.ds(base, _BPW)], idx_v)
            pltpu.async_copy(table_hbm.at[idx_v], rows_v, sem).wait()
            pltpu.sync_copy(rows_v, out_hbm.at[pl.ds(base, _BPW)])

    return k(word_table, idx1, idx2)


_KS = (3, 4, 5)
_PS = tuple(C - K + 1 for K in _KS)
_ACOLS = sum(P * NF for P in _PS)
_BN = 1024


def _sel(K):
    P = C - K + 1
    s = np.zeros((C, P, K), np.float32)
    for p in range(P):
        for k in range(K):
            s[p + k, p, k] = 1.0
    return s


_SELS = tuple(_sel(K) for K in _KS)


def _build_A(W3, W4, W5):
    blocks = []
    for sel, W in zip(_SELS, (W3, W4, W5)):
        blocks.append(
            jnp.einsum("jpk,fdk->jdpf", sel, W).reshape(C * CD, -1))
    return jnp.concatenate(blocks, axis=1)


def _char_body(qc_ref, wemb_ref, ctb_ref, A_ref, bias_ref, out_ref):
    qc = qc_ref[...]
    qt = pltpu.repeat(qc, CV, axis=1)
    col = lax.broadcasted_iota(jnp.int32, (_BN, CV * C), 1)
    oh = (qt == col // C).astype(jnp.float32)
    e = jnp.dot(oh, ctb_ref[...],
                preferred_element_type=jnp.float32)
    y = jnp.dot(e, A_ref[...],
                preferred_element_type=jnp.float32)
    feats = []
    col0 = 0
    for P in _PS:
        m = y[:, col0:col0 + NF]
        for p in range(1, P):
            m = jnp.maximum(m, y[:, col0 + p * NF:col0 + (p + 1) * NF])
        feats.append(m)
        col0 += P * NF
    ce = jnp.concatenate(feats, axis=-1) + bias_ref[...][None, :]
    ce = jnp.maximum(ce, 0.0)
    out_ref[...] = jnp.concatenate([wemb_ref[...], ce], axis=-1)


def _char_call(qc, wemb, ctb, A, bias):
    grid = (N // _BN,)
    return pl.pallas_call(
        _char_body,
        grid=grid,
        in_specs=[
            pl.BlockSpec((_BN, C), lambda i: (i, 0)),
            pl.BlockSpec((_BN, D), lambda i: (i, 0)),
            pl.BlockSpec((CV * C, C * CD), lambda i: (0, 0)),
            pl.BlockSpec((C * CD, _ACOLS), lambda i: (0, 0)),
            pl.BlockSpec((3 * NF,), lambda i: (0,)),
        ],
        out_specs=pl.BlockSpec((_BN, D + 3 * NF), lambda i: (i, 0)),
        out_shape=jax.ShapeDtypeStruct((N, D + 3 * NF), jnp.float32),
    )(qc, wemb, ctb, A, bias)


def kernel(word_table, char_table, W3, b3, W4, b4, W5, b5,
           q1, q2, q1_len, q2_len, q1_char, q2_char):
    idx1 = q1.reshape(-1).astype(jnp.int32)
    idx2 = q2.reshape(-1).astype(jnp.int32)
    wemb1, wemb2 = _sc_gather_call(word_table, idx1, idx2)

    A = _build_A(W3, W4, W5)
    ctb = jnp.einsum("cd,jk->cjkd", char_table,
                     jnp.eye(C, dtype=jnp.float32)).reshape(CV * C, C * CD)
    bias = jnp.concatenate([b3, b4, b5])
    qc1 = q1_char.reshape(N, C).astype(jnp.int32)
    qc2 = q2_char.reshape(N, C).astype(jnp.int32)
    out1 = _char_call(qc1, wemb1, ctb, A, bias)
    out2 = _char_call(qc2, wemb2, ctb, A, bias)
    return (out1.reshape(B, T, D + 3 * NF), out2.reshape(B, T, D + 3 * NF))


# speedup vs baseline: 4.2319x; 1.4001x over previous
import functools

import numpy as np
import jax
import jax.numpy as jnp
from jax import lax
from jax.experimental import pallas as pl
from jax.experimental.pallas import tpu as pltpu
from jax.experimental.pallas import tpu_sc as plsc

B, T, C = 1024, 50, 16
V, D = 1000000, 64
CV, CD = 128, 32
NF = 50
N = B * T


_NC, _NS = 2, 16
_NW = _NC * _NS
_BPW = N // _NW


def _sc_gather_call(word_table, idx1, idx2):
    mesh = plsc.VectorSubcoreMesh(core_axis_name="c", subcore_axis_name="s")

    @functools.partial(
        pl.kernel,
        mesh=mesh,
        compiler_params=pltpu.CompilerParams(use_tc_tiling_on_sc=False),
        out_type=(
            jax.ShapeDtypeStruct((N // 2, 2 * D), jnp.float32),
            jax.ShapeDtypeStruct((N // 2, 2 * D), jnp.float32),
        ),
        scratch_types=[
            pltpu.VMEM((_BPW,), jnp.int32),
            pltpu.VMEM((_BPW, D), jnp.float32),
            pltpu.SemaphoreType.DMA,
        ],
    )
    def k(table_hbm, idx1_hbm, idx2_hbm, out1_hbm, out2_hbm, idx_v, rows_v, sem):
        wid = lax.axis_index("s") * _NC + lax.axis_index("c")
        base = wid * _BPW
        half = wid // (_NW // 2)
        row0 = base - half * (N // 2)
        for idx_hbm, out_hbm in ((idx1_hbm, out1_hbm), (idx2_hbm, out2_hbm)):
            pltpu.sync_copy(idx_hbm.at[pl.ds(base, _BPW)], idx_v)
            pltpu.async_copy(table_hbm.at[idx_v], rows_v, sem).wait()
            pltpu.sync_copy(rows_v,
                            out_hbm.at[pl.ds(row0, _BPW), pl.ds(half * D, D)])

    return k(word_table, idx1, idx2)


_KS = (3, 4, 5)
_PS = tuple(C - K + 1 for K in _KS)
_NFP = 64
_ACOLS = sum(P * _NFP for P in _PS)
_BN = 1024


def _sel(K):
    P = C - K + 1
    s = np.zeros((C, P, K), np.float32)
    for p in range(P):
        for k in range(K):
            s[p + k, p, k] = 1.0
    return s


_SELS = tuple(_sel(K) for K in _KS)


def _build_A(W3, W4, W5):
    blocks = []
    for sel, W in zip(_SELS, (W3, W4, W5)):
        blk = jnp.einsum("jpk,fdk->jdpf", sel, W)
        blk = jnp.pad(blk, ((0, 0), (0, 0), (0, 0), (0, _NFP - NF)))
        blocks.append(blk.reshape(C * CD, -1))
    return jnp.concatenate(blocks, axis=1)


def _char_body(qc_ref, wemb_ref, ctb_ref, A_ref, bias_ref, out_ref):
    qc = qc_ref[...]
    qt = pltpu.repeat(qc, CV, axis=1)
    col = lax.broadcasted_iota(jnp.int32, (_BN, CV * C), 1)
    oh = (qt == col // C).astype(jnp.bfloat16)
    e = jnp.dot(oh, ctb_ref[...],
                preferred_element_type=jnp.float32)
    y = jnp.dot(e.astype(jnp.bfloat16), A_ref[...],
                preferred_element_type=jnp.float32
                ).astype(jnp.bfloat16)
    feats = []
    col0 = 0
    for P in _PS:
        m = y[:, col0:col0 + _NFP]
        for p in range(1, P):
            m = jnp.maximum(m, y[:, col0 + p * _NFP:col0 + (p + 1) * _NFP])
        feats.append(m[:, :NF].astype(jnp.float32))
        col0 += P * _NFP
    ce = jnp.concatenate(feats, axis=-1) + bias_ref[...][None, :]
    ce = jnp.maximum(ce, 0.0)
    wp = wemb_ref[...]
    w = jnp.where(pl.program_id(0) < (N // _BN) // 2,
                  wp[:, :D], wp[:, D:])
    out_ref[...] = jnp.concatenate([w, ce], axis=-1)


def _char_call(qc, wemb, ctb, A, bias):
    grid = (N // _BN,)
    return pl.pallas_call(
        _char_body,
        grid=grid,
        in_specs=[
            pl.BlockSpec((_BN, C), lambda i: (i, 0)),
            pl.BlockSpec((_BN, 2 * D),
                         lambda i: (lax.rem(i, N // _BN // 2), 0)),
            pl.BlockSpec((CV * C, C * CD), lambda i: (0, 0)),
            pl.BlockSpec((C * CD, _ACOLS), lambda i: (0, 0)),
            pl.BlockSpec((3 * NF,), lambda i: (0,)),
        ],
        out_specs=pl.BlockSpec((_BN, D + 3 * NF), lambda i: (i, 0)),
        out_shape=jax.ShapeDtypeStruct((N, D + 3 * NF), jnp.float32),
    )(qc, wemb, ctb, A, bias)


def kernel(word_table, char_table, W3, b3, W4, b4, W5, b5,
           q1, q2, q1_len, q2_len, q1_char, q2_char):
    idx1 = q1.reshape(-1).astype(jnp.int32)
    idx2 = q2.reshape(-1).astype(jnp.int32)
    wemb1, wemb2 = _sc_gather_call(word_table, idx1, idx2)

    A = _build_A(W3, W4, W5).astype(jnp.bfloat16)
    ctb = jnp.einsum("cd,jk->cjkd", char_table,
                     jnp.eye(C, dtype=jnp.float32)).reshape(
                         CV * C, C * CD).astype(jnp.bfloat16)
    bias = jnp.concatenate([b3, b4, b5])
    qc1 = q1_char.reshape(N, C).astype(jnp.int32)
    qc2 = q2_char.reshape(N, C).astype(jnp.int32)
    out1 = _char_call(qc1, wemb1, ctb, A, bias)
    out2 = _char_call(qc2, wemb2, ctb, A, bias)
    return (out1.reshape(B, T, D + 3 * NF), out2.reshape(B, T, D + 3 * NF))
